# Initial kernel scaffold; baseline (speedup 1.0000x reference)
import jax
import jax.numpy as jnp
from jax.experimental import pallas as pl


def kernel(x, edge_index, edge_attr, u, batch, W1, b1, W2, b2):
    raise NotImplementedError("write your pallas kernel here")


# speedup vs baseline: 12.5750x; 12.5750x over previous
import jax
import jax.numpy as jnp
from jax import lax
from jax.experimental import pallas as pl
from jax.experimental.pallas import tpu as pltpu
from jax.experimental.pallas import tpu_sc as plsc

N = 10000
E = 320000
B = 64
DX = 128
DE = 16
DU = 128
H1 = 512
H2 = 256

NC = 2
NS = 16
E_PER_W = E // (NC * NS)
KG = 2000
S_PER_W = E_PER_W // KG


def _sc_edge_kernel(row_hbm, attr_hbm, batch_hbm, out_sum, out_cnt,
                    batch_v, rows_v, attr_v, acc_v, cnt_v, tmp_v, all_sh):
    c = lax.axis_index("c")
    s = lax.axis_index("s")
    wid = s * NC + c

    def zrow(i, _):
        acc_v[i, :] = jnp.zeros((16,), jnp.float32)
        cnt_v[i, :] = jnp.zeros((16,), jnp.float32)
        return 0
    lax.fori_loop(0, B, zrow, 0)

    pltpu.sync_copy(batch_hbm, batch_v)
    ones16 = jnp.ones((16,), jnp.float32)
    base = wid * E_PER_W

    def chunk(si, _):
        off = base + si * KG
        pltpu.sync_copy(row_hbm.at[pl.ds(off, KG)], rows_v)
        pltpu.sync_copy(attr_hbm.at[pl.ds(off, KG)], attr_v)

        def grp(i, _):
            r = rows_v[pl.ds(16 * i, 16)]
            sv = plsc.load_gather(batch_v, [r])
            for kk in range(16):
                sk = sv[kk]
                plsc.addupdate(acc_v.at[sk], attr_v[16 * i + kk, :])
                plsc.addupdate(cnt_v.at[sk], ones16)
            return 0
        lax.fori_loop(0, KG // 16, grp, 0)
        return 0
    lax.fori_loop(0, S_PER_W, chunk, 0)

    pltpu.sync_copy(acc_v, all_sh.at[0].at[s])
    pltpu.sync_copy(cnt_v, all_sh.at[1].at[s])
    plsc.subcore_barrier()

    @pl.when(s == 0)
    def _():
        def red(t, _):
            pltpu.sync_copy(all_sh.at[0].at[t], tmp_v)
            def addrow(i, _):
                acc_v[i, :] = acc_v[i, :] + tmp_v[i, :]
                return 0
            lax.fori_loop(0, B, addrow, 0)
            pltpu.sync_copy(all_sh.at[1].at[t], tmp_v)
            def addrow2(i, _):
                cnt_v[i, :] = cnt_v[i, :] + tmp_v[i, :]
                return 0
            lax.fori_loop(0, B, addrow2, 0)
            return 0
        lax.fori_loop(1, NS, red, 0)
        pltpu.sync_copy(acc_v, out_sum.at[c])
        pltpu.sync_copy(cnt_v, out_cnt.at[c])


def _sc_edge(row, edge_attr, batch32):
    mesh = plsc.VectorSubcoreMesh(core_axis_name="c", subcore_axis_name="s")
    f32 = jnp.float32
    return pl.kernel(
        _sc_edge_kernel,
        out_type=(jax.ShapeDtypeStruct((NC, B, DE), f32),
                  jax.ShapeDtypeStruct((NC, B, DE), f32)),
        mesh=mesh,
        compiler_params=pltpu.CompilerParams(
            needs_layout_passes=False, use_tc_tiling_on_sc=False),
        scratch_types=[
            pltpu.VMEM((N,), jnp.int32),
            pltpu.VMEM((KG,), jnp.int32),
            pltpu.VMEM((KG, DE), f32),
            pltpu.VMEM((B, DE), f32),
            pltpu.VMEM((B, DE), f32),
            pltpu.VMEM((B, DE), f32),
            pltpu.VMEM_SHARED((2, NS, B, DE), f32),
        ],
    )(row, edge_attr, batch32)


NBN = N // 128
NTAIL = N - NBN * 128
NPAD = (NBN + 2) * 128


def _tc_fuse_kernel(x_ref, b_ref, u_ref, w1u_ref, w1x_ref, w1e_ref, b1_ref,
                    w2_ref, b2_ref, es_ref, ec_ref, out_ref):
    f32 = jnp.float32
    bins = lax.broadcasted_iota(jnp.int32, (B, 128), 0)
    ones_n = jnp.ones((128, DE), f32)

    def nstep(j, carry):
        nsum, ncnt = carry
        brow = b_ref[pl.ds(j, 1), :]
        oh = (jnp.broadcast_to(brow, (B, 128)) == bins).astype(f32)
        x_blk = x_ref[pl.ds(j * 128, 128), :]
        nsum = nsum + lax.dot_general(
            oh, x_blk, (((1,), (0,)), ((), ())), preferred_element_type=f32)
        ncnt = ncnt + lax.dot_general(
            oh, ones_n, (((1,), (0,)), ((), ())), preferred_element_type=f32)
        return nsum, ncnt

    nsum, ncnt = lax.fori_loop(
        0, NBN, nstep, (jnp.zeros((B, DX), f32), jnp.zeros((B, DE), f32)))

    btail = b_ref[pl.ds(NBN, 1), pl.ds(0, NTAIL)]
    oh_t = (jnp.broadcast_to(btail, (B, NTAIL))
            == lax.broadcasted_iota(jnp.int32, (B, NTAIL), 0)).astype(f32)
    x_t = x_ref[pl.ds(NBN * 128, NTAIL), :]
    nsum = nsum + lax.dot_general(
        oh_t, x_t, (((1,), (0,)), ((), ())), preferred_element_type=f32)
    ncnt = ncnt + lax.dot_general(
        oh_t, jnp.ones((NTAIL, DE), f32), (((1,), (0,)), ((), ())),
        preferred_element_type=f32)

    nmean = nsum / jnp.maximum(ncnt[:, 0:1], 1.0)

    es = es_ref[0] + es_ref[1]
    ec = ec_ref[0, :, 0:1] + ec_ref[1, :, 0:1]
    emean = es / jnp.maximum(ec, 1.0)

    z = (jnp.dot(u_ref[...], w1u_ref[...], preferred_element_type=f32)
         + jnp.dot(nmean, w1x_ref[...], preferred_element_type=f32)
         + jnp.dot(emean, w1e_ref[...], preferred_element_type=f32)
         + b1_ref[...])
    h = jnp.maximum(z, 0.0)
    out_ref[...] = jnp.dot(h, w2_ref[...], preferred_element_type=f32) \
        + b2_ref[...]


def _tc_fuse(x, bp, u, w1u, w1x, w1e, b1, w2, b2, esum, ecnt):
    return pl.pallas_call(
        _tc_fuse_kernel,
        out_shape=jax.ShapeDtypeStruct((B, H2), jnp.float32),
    )(x, bp, u, w1u, w1x, w1e, b1, w2, b2, esum, ecnt)


@jax.jit
def kernel(x, edge_index, edge_attr, u, batch, W1, b1, W2, b2):
    row = edge_index[0].astype(jnp.int32)
    batch32 = batch.astype(jnp.int32)

    esum, ecnt = _sc_edge(row, edge_attr, batch32)

    bp = jnp.pad(batch32, (0, NPAD - N), constant_values=-1).reshape(-1, 128)
    w1u = W1[:DU]
    w1x = W1[DU:DU + DX]
    w1e = W1[DU + DX:]
    return _tc_fuse(x, bp, u, w1u, w1x, w1e, b1.reshape(1, H1), W2,
                    b2.reshape(1, H2), esum, ecnt)
